# baseline (speedup 1.0000x reference)
import jax
import jax.numpy as jnp
from jax.experimental import pallas as pl
from jax.experimental.pallas import tpu as pltpu

_VMEM_LIMIT_BYTES = 48 * 1024 * 1024


def kernel(x, w1, w2):
    B, C, H, W = x.shape
    HW = H * W
    hidden = w1.shape[0]
    inv_hw = 1.0 / float(HW)

    x3 = x.reshape(B, C, HW)

    NB = 16
    assert B % NB == 0
    CS = C // 2

    w1_lo = w1[:, :CS]
    w1_hi = w1[:, CS:]
    w2_lo = w2[:CS]
    w2_hi = w2[CS:]

    def body(xlo_ref, xhi_ref, w1lo_ref, w1hi_ref, w2lo_ref, w2hi_ref, o_ref):
        xlo = xlo_ref[...].reshape(NB * CS, HW)
        xhi = xhi_ref[...].reshape(NB * CS, HW)
        plo = jnp.sum(xlo, axis=-1, keepdims=True,
                      dtype=jnp.float32) * inv_hw
        phi = jnp.sum(xhi, axis=-1, keepdims=True,
                      dtype=jnp.float32) * inv_hw
        glo_parts = []
        ghi_parts = []
        for b in range(NB):
            pb_lo = plo[b * CS:(b + 1) * CS]
            pb_hi = phi[b * CS:(b + 1) * CS]
            hb = jnp.maximum(
                jnp.dot(w1lo_ref[...], pb_lo,
                        preferred_element_type=jnp.float32)
                + jnp.dot(w1hi_ref[...], pb_hi,
                          preferred_element_type=jnp.float32), 0.0)
            glo_parts.append(jnp.dot(w2lo_ref[...], hb,
                                     preferred_element_type=jnp.float32))
            ghi_parts.append(jnp.dot(w2hi_ref[...], hb,
                                     preferred_element_type=jnp.float32))
        glo = jax.nn.sigmoid(jnp.concatenate(glo_parts, axis=0))
        ghi = jax.nn.sigmoid(jnp.concatenate(ghi_parts, axis=0))
        o_ref[:, :CS, :] = (xlo * glo).reshape(NB, CS, HW).astype(o_ref.dtype)
        o_ref[:, CS:, :] = (xhi * ghi).reshape(NB, CS, HW).astype(o_ref.dtype)

    out = pl.pallas_call(
        body,
        out_shape=jax.ShapeDtypeStruct((B, C, HW), x.dtype),
        grid=(B // NB,),
        in_specs=[
            pl.BlockSpec((NB, CS, HW), lambda b: (b, 0, 0)),
            pl.BlockSpec((NB, CS, HW), lambda b: (b, 1, 0)),
            pl.BlockSpec((hidden, CS), lambda b: (0, 0)),
            pl.BlockSpec((hidden, CS), lambda b: (0, 0)),
            pl.BlockSpec((CS, hidden), lambda b: (0, 0)),
            pl.BlockSpec((CS, hidden), lambda b: (0, 0)),
        ],
        out_specs=pl.BlockSpec((NB, C, HW), lambda b: (b, 0, 0)),
        compiler_params=pltpu.CompilerParams(
            dimension_semantics=("parallel",),
            vmem_limit_bytes=_VMEM_LIMIT_BYTES),
    )(x3, x3, w1_lo, w1_hi, w2_lo, w2_hi)
    return out.reshape(B, C, H, W)


# speedup vs baseline: 1.0281x; 1.0281x over previous
import jax
import jax.numpy as jnp
from jax.experimental import pallas as pl
from jax.experimental.pallas import tpu as pltpu

_VMEM_LIMIT_BYTES = 48 * 1024 * 1024


def kernel(x, w1, w2):
    B, C, H, W = x.shape
    HW = H * W
    hidden = w1.shape[0]
    inv_hw = 1.0 / float(HW)

    x3 = x.reshape(B, C, HW)
    w1t = jnp.transpose(w1).astype(jnp.float32) * inv_hw
    w2t = jnp.transpose(w2).astype(jnp.float32)

    NB = 16
    assert B % NB == 0

    def body(x_ref, w1t_ref, w2t_ref, o_ref):
        xb = x_ref[...]
        pooled = jnp.sum(xb, axis=-1, dtype=jnp.float32)
        h = jnp.maximum(
            jnp.dot(pooled, w1t_ref[...],
                    preferred_element_type=jnp.float32), 0.0)
        gate = jax.nn.sigmoid(
            jnp.dot(h, w2t_ref[...],
                    preferred_element_type=jnp.float32))
        gate_cols = gate.T.astype(xb.dtype)
        for b in range(NB):
            o_ref[b] = (xb[b] * gate_cols[:, b:b + 1]).astype(o_ref.dtype)

    out = pl.pallas_call(
        body,
        out_shape=jax.ShapeDtypeStruct((B, C, HW), x.dtype),
        grid=(B // NB,),
        in_specs=[
            pl.BlockSpec((NB, C, HW), lambda b: (b, 0, 0)),
            pl.BlockSpec((C, hidden), lambda b: (0, 0)),
            pl.BlockSpec((hidden, C), lambda b: (0, 0)),
        ],
        out_specs=pl.BlockSpec((NB, C, HW), lambda b: (b, 0, 0)),
        compiler_params=pltpu.CompilerParams(
            dimension_semantics=("parallel",),
            vmem_limit_bytes=_VMEM_LIMIT_BYTES),
    )(x3, w1t, w2t)
    return out.reshape(B, C, H, W)
